# baseline (speedup 1.0000x reference)
import jax
import jax.numpy as jnp
from jax.experimental import pallas as pl
from jax.experimental.pallas import tpu as pltpu

_CHUNKS = 16


def kernel(x, embedding):
    seq_len = x.shape[1]
    d_model = embedding.shape[-1]
    chunks = _CHUNKS
    while seq_len % chunks != 0 and chunks > 1:
        chunks //= 2
    rows = seq_len // chunks

    def body(emb_hbm, out_hbm, vmem, in_sems, out_sems):
        for k in range(chunks):
            pltpu.make_async_copy(
                emb_hbm.at[0, pl.ds(k * rows, rows), :], vmem.at[k], in_sems.at[k]
            ).start()
        for k in range(chunks):
            pltpu.make_async_copy(
                emb_hbm.at[0, pl.ds(k * rows, rows), :], vmem.at[k], in_sems.at[k]
            ).wait()
            pltpu.make_async_copy(
                vmem.at[k], out_hbm.at[0, pl.ds(k * rows, rows), :], out_sems.at[k]
            ).start()
        for k in range(chunks):
            pltpu.make_async_copy(
                vmem.at[k], out_hbm.at[0, pl.ds(k * rows, rows), :], out_sems.at[k]
            ).wait()

    return pl.pallas_call(
        body,
        in_specs=[pl.BlockSpec(memory_space=pl.ANY)],
        out_specs=pl.BlockSpec(memory_space=pl.ANY),
        out_shape=jax.ShapeDtypeStruct((1, seq_len, d_model), embedding.dtype),
        scratch_shapes=[
            pltpu.VMEM((chunks, rows, d_model), embedding.dtype),
            pltpu.SemaphoreType.DMA((chunks,)),
            pltpu.SemaphoreType.DMA((chunks,)),
        ],
    )(embedding)


# speedup vs baseline: 1.0185x; 1.0185x over previous
import jax
import jax.numpy as jnp
from jax.experimental import pallas as pl
from jax.experimental.pallas import tpu as pltpu

_CHUNKS = 4


def kernel(x, embedding):
    seq_len = x.shape[1]
    d_model = embedding.shape[-1]
    chunks = _CHUNKS
    while seq_len % chunks != 0 and chunks > 1:
        chunks //= 2
    rows = seq_len // chunks

    def body(emb_hbm, out_hbm, vmem, in_sems, out_sems):
        for k in range(chunks):
            pltpu.make_async_copy(
                emb_hbm.at[0, pl.ds(k * rows, rows), :], vmem.at[k], in_sems.at[k]
            ).start()
        for k in range(chunks):
            pltpu.make_async_copy(
                emb_hbm.at[0, pl.ds(k * rows, rows), :], vmem.at[k], in_sems.at[k]
            ).wait()
            pltpu.make_async_copy(
                vmem.at[k], out_hbm.at[0, pl.ds(k * rows, rows), :], out_sems.at[k]
            ).start()
        for k in range(chunks):
            pltpu.make_async_copy(
                vmem.at[k], out_hbm.at[0, pl.ds(k * rows, rows), :], out_sems.at[k]
            ).wait()

    return pl.pallas_call(
        body,
        in_specs=[pl.BlockSpec(memory_space=pl.ANY)],
        out_specs=pl.BlockSpec(memory_space=pl.ANY),
        out_shape=jax.ShapeDtypeStruct((1, seq_len, d_model), embedding.dtype),
        scratch_shapes=[
            pltpu.VMEM((chunks, rows, d_model), embedding.dtype),
            pltpu.SemaphoreType.DMA((chunks,)),
            pltpu.SemaphoreType.DMA((chunks,)),
        ],
    )(embedding)
